# initial kernel scaffold (unmeasured)
import jax
import jax.numpy as jnp
from jax import lax
from jax.experimental import pallas as pl
from jax.experimental.pallas import tpu as pltpu

N_DEV = 16
M = 4096
N = 2048
CHUNK = M // N_DEV
N_HOPS = 2 * (N_DEV - 1)


def kernel(x, w_mat):
    def body(x_ref, w_ref, out_ref, comm_ref, send_sems, recv_sems, credit_sem):
        my = lax.axis_index("i")
        left = (my - 1) % N_DEV
        right = (my + 1) % N_DEV

        barrier_sem = pltpu.get_barrier_semaphore()
        for nbr in (left, right):
            pl.semaphore_signal(
                barrier_sem, inc=1,
                device_id=(nbr,), device_id_type=pl.DeviceIdType.MESH,
            )
        pl.semaphore_wait(barrier_sem, 2)

        out_ref[...] = jnp.dot(
            x_ref[...], w_ref[...], preferred_element_type=jnp.float32
        )

        for h in range(N_HOPS):
            slot = h % 2
            if h < N_DEV - 1:
                s = h
                c_send = (my - s) % N_DEV
                c_recv = (my - s - 1) % N_DEV
            else:
                s = h - (N_DEV - 1)
                c_send = (my + 1 - s) % N_DEV
                c_recv = (my - s) % N_DEV

            if h >= 2:
                pl.semaphore_wait(credit_sem, 1)

            rdma = pltpu.make_async_remote_copy(
                src_ref=out_ref.at[pl.ds(c_send * CHUNK, CHUNK), :],
                dst_ref=comm_ref.at[slot],
                send_sem=send_sems.at[slot],
                recv_sem=recv_sems.at[slot],
                device_id=(right,),
                device_id_type=pl.DeviceIdType.MESH,
            )
            rdma.start()
            rdma.wait()

            if h < N_DEV - 1:
                out_ref[pl.ds(c_recv * CHUNK, CHUNK), :] = (
                    out_ref[pl.ds(c_recv * CHUNK, CHUNK), :] + comm_ref[slot]
                )
            else:
                out_ref[pl.ds(c_recv * CHUNK, CHUNK), :] = comm_ref[slot]

            if h < N_HOPS - 2:
                pl.semaphore_signal(
                    credit_sem, inc=1,
                    device_id=(left,), device_id_type=pl.DeviceIdType.MESH,
                )

        y = out_ref[...]
        scale = jnp.max(jnp.abs(y)) / 127.0
        q = jnp.clip(jnp.round(y / scale), -127.0, 127.0)
        out_ref[...] = q * scale

    return pl.pallas_call(
        body,
        out_shape=jax.ShapeDtypeStruct((M, N), jnp.float32),
        in_specs=[
            pl.BlockSpec(memory_space=pltpu.VMEM),
            pl.BlockSpec(memory_space=pltpu.VMEM),
        ],
        out_specs=pl.BlockSpec(memory_space=pltpu.VMEM),
        scratch_shapes=[
            pltpu.VMEM((2, CHUNK, N), jnp.float32),
            pltpu.SemaphoreType.DMA((2,)),
            pltpu.SemaphoreType.DMA((2,)),
            pltpu.SemaphoreType.REGULAR,
        ],
        compiler_params=pltpu.CompilerParams(collective_id=0),
    )(x, w_mat)


# baseline (device time: 789645 ns/iter reference)
import jax
import jax.numpy as jnp
from jax import lax
from jax.experimental import pallas as pl
from jax.experimental.pallas import tpu as pltpu

N_DEV = 16
M = 4096
N = 2048
CHUNK = M // N_DEV
N_HOPS = 2 * (N_DEV - 1)


def kernel(x, w_mat):
    def body(x_ref, w_ref, out_ref, comm_ref, send_sems, recv_sems, credit_sem):
        my = lax.axis_index("i")
        left = (my - 1) % N_DEV
        right = (my + 1) % N_DEV

        barrier_sem = pltpu.get_barrier_semaphore()
        for nbr in (left, right):
            pl.semaphore_signal(
                barrier_sem, inc=1,
                device_id=(nbr,), device_id_type=pl.DeviceIdType.MESH,
            )
        pl.semaphore_wait(barrier_sem, 2)

        for c in range(N_DEV):
            out_ref[c * CHUNK:(c + 1) * CHUNK, :] = jnp.dot(
                x_ref[c * CHUNK:(c + 1) * CHUNK, :], w_ref[...],
                preferred_element_type=jnp.float32,
            )

        for h in range(N_HOPS):
            slot = h % 2
            if h < N_DEV - 1:
                s = h
                c_send = (my - s) % N_DEV
                c_recv = (my - s - 1) % N_DEV
            else:
                s = h - (N_DEV - 1)
                c_send = (my + 1 - s) % N_DEV
                c_recv = (my - s) % N_DEV

            if h >= 2:
                pl.semaphore_wait(credit_sem, 1)

            rdma = pltpu.make_async_remote_copy(
                src_ref=out_ref.at[pl.ds(c_send * CHUNK, CHUNK), :],
                dst_ref=comm_ref.at[slot],
                send_sem=send_sems.at[slot],
                recv_sem=recv_sems.at[slot],
                device_id=(right,),
                device_id_type=pl.DeviceIdType.MESH,
            )
            rdma.start()
            rdma.wait()

            if h < N_DEV - 1:
                out_ref[pl.ds(c_recv * CHUNK, CHUNK), :] = (
                    out_ref[pl.ds(c_recv * CHUNK, CHUNK), :] + comm_ref[slot]
                )
            else:
                out_ref[pl.ds(c_recv * CHUNK, CHUNK), :] = comm_ref[slot]

            if h < N_HOPS - 2:
                pl.semaphore_signal(
                    credit_sem, inc=1,
                    device_id=(left,), device_id_type=pl.DeviceIdType.MESH,
                )

        amax = jnp.float32(0.0)
        for c in range(N_DEV):
            amax = jnp.maximum(
                amax, jnp.max(jnp.abs(out_ref[c * CHUNK:(c + 1) * CHUNK, :]))
            )
        scale = amax / 127.0
        for c in range(N_DEV):
            y = out_ref[c * CHUNK:(c + 1) * CHUNK, :]
            q = jnp.clip(jnp.round(y / scale), -127.0, 127.0)
            out_ref[c * CHUNK:(c + 1) * CHUNK, :] = q * scale

    return pl.pallas_call(
        body,
        out_shape=jax.ShapeDtypeStruct((M, N), jnp.float32),
        in_specs=[
            pl.BlockSpec(memory_space=pltpu.VMEM),
            pl.BlockSpec(memory_space=pltpu.VMEM),
        ],
        out_specs=pl.BlockSpec(memory_space=pltpu.VMEM),
        scratch_shapes=[
            pltpu.VMEM((2, CHUNK, N), jnp.float32),
            pltpu.SemaphoreType.DMA((2,)),
            pltpu.SemaphoreType.DMA((2,)),
            pltpu.SemaphoreType.REGULAR,
        ],
        compiler_params=pltpu.CompilerParams(
            collective_id=0,
            vmem_limit_bytes=60 * 1024 * 1024,
        ),
    )(x, w_mat)


# device time: 488817 ns/iter; 1.6154x vs baseline; 1.6154x over previous
import jax
import jax.numpy as jnp
from jax import lax
from jax.experimental import pallas as pl
from jax.experimental.pallas import tpu as pltpu

N_DEV = 16
M = 4096
N = 2048
CHUNK = M // N_DEV
N_STEPS = N_DEV - 1

N_RINGS = 4
ROWS = CHUNK // N_RINGS
RING_DIR = (1, -1, 1, -1)
RING_OFF = (0, 128, 64, 192)

_MESH = pl.DeviceIdType.MESH


def kernel(x, w_mat):
    def body(x_ref, w_ref, out_ref, comm_ref, send_sems, recv_sems, credit_sems):
        my = lax.axis_index("i")
        left = (my - 1) % N_DEV
        right = (my + 1) % N_DEV

        barrier_sem = pltpu.get_barrier_semaphore()
        for nbr in (left, right):
            pl.semaphore_signal(barrier_sem, inc=1, device_id=(nbr,),
                                device_id_type=_MESH)
        pl.semaphore_wait(barrier_sem, 2)

        for c in range(N_DEV):
            out_ref[c * CHUNK:(c + 1) * CHUNK, :] = jnp.dot(
                x_ref[c * CHUNK:(c + 1) * CHUNK, :], w_ref[...],
                preferred_element_type=jnp.float32,
            )

        def make_rdma(r, slot, c_send, tgt):
            return pltpu.make_async_remote_copy(
                src_ref=out_ref.at[pl.ds(c_send * CHUNK + RING_OFF[r], ROWS), :],
                dst_ref=comm_ref.at[r, slot],
                send_sem=send_sems.at[r, slot],
                recv_sem=recv_sems.at[r, slot],
                device_id=(tgt,),
                device_id_type=_MESH,
            )

        def rs_step(s, carry):
            slot = lax.rem(s, 2)
            rdmas = []
            for r in range(N_RINGS):
                if RING_DIR[r] == 1:
                    c_send = (my - s) % N_DEV
                    tgt = right
                else:
                    c_send = (my + s) % N_DEV
                    tgt = left
                rdma = make_rdma(r, slot, c_send, tgt)

                @pl.when(s >= 2)
                def _(r=r):
                    pl.semaphore_wait(credit_sems.at[r], 1)

                rdma.start()
                rdmas.append(rdma)
            for r in range(N_RINGS):
                rdmas[r].wait()
                if RING_DIR[r] == 1:
                    c_recv = (my - s - 1) % N_DEV
                    upstream = left
                else:
                    c_recv = (my + s + 1) % N_DEV
                    upstream = right
                dst = pl.ds(c_recv * CHUNK + RING_OFF[r], ROWS)
                out_ref[dst, :] = out_ref[dst, :] + comm_ref[r, slot]
                pl.semaphore_signal(credit_sems.at[r], inc=1,
                                    device_id=(upstream,), device_id_type=_MESH)
            return carry

        lax.fori_loop(0, N_STEPS, rs_step, 0)

        own_r = pl.ds(((my + 1) % N_DEV) * CHUNK, 2 * ROWS)
        own_l = pl.ds(((my - 1) % N_DEV) * CHUNK + 2 * ROWS, 2 * ROWS)
        amax0 = jnp.maximum(jnp.max(jnp.abs(out_ref[own_r, :])),
                            jnp.max(jnp.abs(out_ref[own_l, :])))

        def ag_step(s, amax):
            slot = lax.rem(s + 1, 2)
            rdmas = []
            for r in range(N_RINGS):
                if RING_DIR[r] == 1:
                    c_send = (my + 1 - s) % N_DEV
                    tgt = right
                else:
                    c_send = (my - 1 + s) % N_DEV
                    tgt = left
                rdma = make_rdma(r, slot, c_send, tgt)
                pl.semaphore_wait(credit_sems.at[r], 1)
                rdma.start()
                rdmas.append(rdma)
            for r in range(N_RINGS):
                rdmas[r].wait()
                if RING_DIR[r] == 1:
                    c_recv = (my - s) % N_DEV
                    upstream = left
                else:
                    c_recv = (my + s) % N_DEV
                    upstream = right
                val = comm_ref[r, slot]
                out_ref[pl.ds(c_recv * CHUNK + RING_OFF[r], ROWS), :] = val
                amax = jnp.maximum(amax, jnp.max(jnp.abs(val)))

                @pl.when(s <= N_STEPS - 3)
                def _(r=r, upstream=upstream):
                    pl.semaphore_signal(credit_sems.at[r], inc=1,
                                        device_id=(upstream,),
                                        device_id_type=_MESH)
            return amax

        amax = lax.fori_loop(0, N_STEPS, ag_step, amax0)

        scale = amax / 127.0
        for c in range(N_DEV):
            y = out_ref[c * CHUNK:(c + 1) * CHUNK, :]
            q = jnp.clip(jnp.round(y / scale), -127.0, 127.0)
            out_ref[c * CHUNK:(c + 1) * CHUNK, :] = q * scale

    return pl.pallas_call(
        body,
        out_shape=jax.ShapeDtypeStruct((M, N), jnp.float32),
        in_specs=[
            pl.BlockSpec(memory_space=pltpu.VMEM),
            pl.BlockSpec(memory_space=pltpu.VMEM),
        ],
        out_specs=pl.BlockSpec(memory_space=pltpu.VMEM),
        scratch_shapes=[
            pltpu.VMEM((N_RINGS, 2, ROWS, N), jnp.float32),
            pltpu.SemaphoreType.DMA((N_RINGS, 2)),
            pltpu.SemaphoreType.DMA((N_RINGS, 2)),
            pltpu.SemaphoreType.REGULAR((N_RINGS,)),
        ],
        compiler_params=pltpu.CompilerParams(
            collective_id=0,
            vmem_limit_bytes=60 * 1024 * 1024,
        ),
    )(x, w_mat)


# device time: 391899 ns/iter; 2.0149x vs baseline; 1.2473x over previous
import jax
import jax.numpy as jnp
from jax import lax
from jax.experimental import pallas as pl
from jax.experimental.pallas import tpu as pltpu

N_DEV = 16
M = 4096
N = 2048
CHUNK = M // N_DEV
N_HOPS = 2 * (N_DEV - 1)

N_RINGS = 4
ROWS = CHUNK // N_RINGS
RING_DIR = (1, -1, 1, -1)
RING_OFF = (0, 128, 64, 192)

_MESH = pl.DeviceIdType.MESH


def kernel(x, w_mat):
    def body(x_ref, w_ref, out_ref, comm_ref, send_sems, recv_sems, credit_sems):
        my = lax.axis_index("i")
        left = (my - 1) % N_DEV
        right = (my + 1) % N_DEV

        barrier_sem = pltpu.get_barrier_semaphore()
        for nbr in (left, right):
            pl.semaphore_signal(barrier_sem, inc=1, device_id=(nbr,),
                                device_id_type=_MESH)
        pl.semaphore_wait(barrier_sem, 2)

        def gemm_chunk(c):
            rows = pl.ds(c * CHUNK, CHUNK)
            out_ref[rows, :] = jnp.dot(
                x_ref[rows, :], w_ref[...],
                preferred_element_type=jnp.float32,
            )

        def make_rdma(r, slot, h):
            if RING_DIR[r] == 1:
                c_send = (my - h) % N_DEV
                tgt = right
            else:
                c_send = (my + h) % N_DEV
                tgt = left
            return pltpu.make_async_remote_copy(
                src_ref=out_ref.at[pl.ds(c_send * CHUNK + RING_OFF[r], ROWS), :],
                dst_ref=comm_ref.at[r, slot],
                send_sem=send_sems.at[r, slot],
                recv_sem=recv_sems.at[r, slot],
                device_id=(tgt,),
                device_id_type=_MESH,
            )

        gemm_chunk(my)
        for r in range(N_RINGS):
            make_rdma(r, 0, 0).start()
        for j in range(1, N_DEV):
            gemm_chunk((my + j) % N_DEV)

        def step(h, amax):
            slot_prev = lax.rem(h + 1, 2)
            slot_cur = lax.rem(h, 2)
            for r in range(N_RINGS):
                prev = make_rdma(r, slot_prev, h - 1)
                prev.wait()
                if RING_DIR[r] == 1:
                    c_recv = (my - h) % N_DEV
                    upstream = left
                else:
                    c_recv = (my + h) % N_DEV
                    upstream = right
                dst = pl.ds(c_recv * CHUNK + RING_OFF[r], ROWS)
                val = comm_ref[r, slot_prev]
                new = jnp.where(h <= N_DEV - 1, out_ref[dst, :] + val, val)
                out_ref[dst, :] = new
                amax = jnp.where(h >= N_DEV - 1,
                                 jnp.maximum(amax, jnp.max(jnp.abs(new))),
                                 amax)

                @pl.when(h - 1 <= N_HOPS - 3)
                def _(r=r, upstream=upstream):
                    pl.semaphore_signal(credit_sems.at[r], inc=1,
                                        device_id=(upstream,),
                                        device_id_type=_MESH)

                @pl.when(h >= 2)
                def _(r=r):
                    pl.semaphore_wait(credit_sems.at[r], 1)

                make_rdma(r, slot_cur, h).start()
            return amax

        amax = lax.fori_loop(1, N_HOPS, step, jnp.float32(0.0))

        slot_prev = (N_HOPS - 1) % 2
        for r in range(N_RINGS):
            prev = make_rdma(r, slot_prev, N_HOPS - 1)
            prev.wait()
            c_recv = (my - N_HOPS) % N_DEV if RING_DIR[r] == 1 \
                else (my + N_HOPS) % N_DEV
            val = comm_ref[r, slot_prev]
            out_ref[pl.ds(c_recv * CHUNK + RING_OFF[r], ROWS), :] = val
            amax = jnp.maximum(amax, jnp.max(jnp.abs(val)))

        scale = amax / 127.0
        for c in range(N_DEV):
            y = out_ref[c * CHUNK:(c + 1) * CHUNK, :]
            q = jnp.clip(jnp.round(y / scale), -127.0, 127.0)
            out_ref[c * CHUNK:(c + 1) * CHUNK, :] = q * scale

    return pl.pallas_call(
        body,
        out_shape=jax.ShapeDtypeStruct((M, N), jnp.float32),
        in_specs=[
            pl.BlockSpec(memory_space=pltpu.VMEM),
            pl.BlockSpec(memory_space=pltpu.VMEM),
        ],
        out_specs=pl.BlockSpec(memory_space=pltpu.VMEM),
        scratch_shapes=[
            pltpu.VMEM((N_RINGS, 2, ROWS, N), jnp.float32),
            pltpu.SemaphoreType.DMA((N_RINGS, 2)),
            pltpu.SemaphoreType.DMA((N_RINGS, 2)),
            pltpu.SemaphoreType.REGULAR((N_RINGS,)),
        ],
        compiler_params=pltpu.CompilerParams(
            collective_id=0,
            vmem_limit_bytes=60 * 1024 * 1024,
        ),
    )(x, w_mat)


# device time: 280995 ns/iter; 2.8102x vs baseline; 1.3947x over previous
import jax
import jax.numpy as jnp
from jax import lax
from jax.experimental import pallas as pl
from jax.experimental.pallas import tpu as pltpu

N_DEV = 16
M = 4096
N = 2048
CHUNK = M // N_DEV
N_HOPS = 2 * (N_DEV - 1)
N_BF = 4

N_RINGS = 4
ROWS = CHUNK // N_RINGS
RING_DIR = (1, -1, 1, -1)
RING_OFF = (0, 128, 64, 192)

_MESH = pl.DeviceIdType.MESH


def kernel(x, w_mat):
    def body(x_ref, w_ref, out_ref, comm_ref, commq_ref, q_ref,
             bf_send_ref, bf_recv_ref, send_sems, recv_sems, credit_sems,
             bf_send_sems, bf_recv_sems):
        my = lax.axis_index("i")
        left = (my - 1) % N_DEV
        right = (my + 1) % N_DEV

        barrier_sem = pltpu.get_barrier_semaphore()
        for nbr in (left, right):
            pl.semaphore_signal(barrier_sem, inc=1, device_id=(nbr,),
                                device_id_type=_MESH)
        pl.semaphore_wait(barrier_sem, 2)

        def gemm_chunk(c):
            rows = pl.ds(c * CHUNK, CHUNK)
            out_ref[rows, :] = jnp.dot(
                x_ref[rows, :], w_ref[...],
                preferred_element_type=jnp.float32,
            )

        def ring_ids(r, h):
            if RING_DIR[r] == 1:
                return (my - h) % N_DEV, (my - h - 1) % N_DEV, right, left
            return (my + h) % N_DEV, (my + h + 1) % N_DEV, left, right

        def strip(c, r):
            return pl.ds(c * CHUNK + RING_OFF[r], ROWS)

        def rs_rdma(r, slot, h):
            c_send, _, tgt, _ = ring_ids(r, h)
            return pltpu.make_async_remote_copy(
                src_ref=out_ref.at[strip(c_send, r), :],
                dst_ref=comm_ref.at[r, slot],
                send_sem=send_sems.at[r, slot],
                recv_sem=recv_sems.at[r, slot],
                device_id=(tgt,), device_id_type=_MESH,
            )

        def ag_rdma(r, slot, h):
            c_send, _, tgt, _ = ring_ids(r, h)
            return pltpu.make_async_remote_copy(
                src_ref=q_ref.at[strip(c_send, r), :],
                dst_ref=commq_ref.at[r, slot],
                send_sem=send_sems.at[r, slot],
                recv_sem=recv_sems.at[r, slot],
                device_id=(tgt,), device_id_type=_MESH,
            )

        gemm_chunk(my)
        for r in range(N_RINGS):
            rs_rdma(r, 0, 0).start()
        for j in range(1, N_DEV):
            gemm_chunk((my + j) % N_DEV)

        def rs_step(h, carry):
            slot_prev = lax.rem(h + 1, 2)
            slot_cur = lax.rem(h, 2)
            for r in range(N_RINGS):
                prev = rs_rdma(r, slot_prev, h - 1)
                prev.wait()
                _, c_recv, _, upstream = ring_ids(r, h - 1)
                dst = strip(c_recv, r)
                out_ref[dst, :] = out_ref[dst, :] + comm_ref[r, slot_prev]
                pl.semaphore_signal(credit_sems.at[r], inc=1,
                                    device_id=(upstream,),
                                    device_id_type=_MESH)

                @pl.when(h >= 2)
                def _(r=r):
                    pl.semaphore_wait(credit_sems.at[r], 1)

                rs_rdma(r, slot_cur, h).start()
            return carry

        lax.fori_loop(1, N_DEV - 1, rs_step, 0)

        for r in range(N_RINGS):
            prev = rs_rdma(r, (N_DEV - 2) % 2, N_DEV - 2)
            prev.wait()
            _, c_recv, _, upstream = ring_ids(r, N_DEV - 2)
            dst = strip(c_recv, r)
            out_ref[dst, :] = out_ref[dst, :] + comm_ref[r, (N_DEV - 2) % 2]
            pl.semaphore_signal(credit_sems.at[r], inc=1,
                                device_id=(upstream,), device_id_type=_MESH)

        own_r = pl.ds(((my + 1) % N_DEV) * CHUNK, 2 * ROWS)
        own_l = pl.ds(((my - 1) % N_DEV) * CHUNK + 2 * ROWS, 2 * ROWS)
        amax = jnp.maximum(jnp.max(jnp.abs(out_ref[own_r, :])),
                           jnp.max(jnp.abs(out_ref[own_l, :])))
        for k in range(N_BF):
            partner = my ^ (1 << k)
            bf_send_ref[...] = jnp.full((8, 128), amax, jnp.float32)
            bf = pltpu.make_async_remote_copy(
                src_ref=bf_send_ref,
                dst_ref=bf_recv_ref.at[k],
                send_sem=bf_send_sems.at[k],
                recv_sem=bf_recv_sems.at[k],
                device_id=(partner,), device_id_type=_MESH,
            )
            bf.start()
            bf.wait()
            amax = jnp.maximum(amax, bf_recv_ref[k, 0, 0])
        scale = amax / 127.0

        for r in range(N_RINGS):
            c_own = (my + 1) % N_DEV if RING_DIR[r] == 1 else (my - 1) % N_DEV
            dst = strip(c_own, r)
            qf = jnp.clip(jnp.round(out_ref[dst, :] / scale), -127.0, 127.0)
            q_ref[dst, :] = qf.astype(jnp.int8)
            out_ref[dst, :] = qf * scale
        for r in range(N_RINGS):
            pl.semaphore_wait(credit_sems.at[r], 1)
            ag_rdma(r, (N_DEV - 1) % 2, N_DEV - 1).start()

        def ag_step(h, carry):
            slot_prev = lax.rem(h + 1, 2)
            slot_cur = lax.rem(h, 2)
            for r in range(N_RINGS):
                prev = ag_rdma(r, slot_prev, h - 1)
                prev.wait()
                _, c_recv, _, upstream = ring_ids(r, h - 1)
                dst = strip(c_recv, r)
                val = commq_ref[r, slot_prev]
                q_ref[dst, :] = val
                out_ref[dst, :] = val.astype(jnp.float32) * scale

                @pl.when(h - 1 <= N_HOPS - 3)
                def _(r=r, upstream=upstream):
                    pl.semaphore_signal(credit_sems.at[r], inc=1,
                                        device_id=(upstream,),
                                        device_id_type=_MESH)

                pl.semaphore_wait(credit_sems.at[r], 1)
                ag_rdma(r, slot_cur, h).start()
            return carry

        lax.fori_loop(N_DEV, N_HOPS, ag_step, 0)

        for r in range(N_RINGS):
            prev = ag_rdma(r, (N_HOPS - 1) % 2, N_HOPS - 1)
            prev.wait()
            _, c_recv, _, _ = ring_ids(r, N_HOPS - 1)
            val = commq_ref[r, (N_HOPS - 1) % 2]
            out_ref[strip(c_recv, r), :] = val.astype(jnp.float32) * scale

    return pl.pallas_call(
        body,
        out_shape=jax.ShapeDtypeStruct((M, N), jnp.float32),
        in_specs=[
            pl.BlockSpec(memory_space=pltpu.VMEM),
            pl.BlockSpec(memory_space=pltpu.VMEM),
        ],
        out_specs=pl.BlockSpec(memory_space=pltpu.VMEM),
        scratch_shapes=[
            pltpu.VMEM((N_RINGS, 2, ROWS, N), jnp.float32),
            pltpu.VMEM((N_RINGS, 2, ROWS, N), jnp.int8),
            pltpu.VMEM((M, N), jnp.int8),
            pltpu.VMEM((8, 128), jnp.float32),
            pltpu.VMEM((N_BF, 8, 128), jnp.float32),
            pltpu.SemaphoreType.DMA((N_RINGS, 2)),
            pltpu.SemaphoreType.DMA((N_RINGS, 2)),
            pltpu.SemaphoreType.REGULAR((N_RINGS,)),
            pltpu.SemaphoreType.DMA((N_BF,)),
            pltpu.SemaphoreType.DMA((N_BF,)),
        ],
        compiler_params=pltpu.CompilerParams(
            collective_id=0,
            vmem_limit_bytes=60 * 1024 * 1024,
        ),
    )(x, w_mat)
